# baseline (speedup 1.0000x reference)
import functools

import jax
import jax.numpy as jnp
from jax import lax
from jax.experimental import pallas as pl
from jax.experimental.pallas import tpu as pltpu
from jax.experimental.pallas import tpu_sc as plsc

E = 64
TOPK = 2
CAP = 128
NC = 2
NS = 16
NW = NC * NS


def _router_body(x_ref, wg_ref, tok_slot_ref, gate_slot_ref, s0_ref, s1_ref):
    T = x_ref.shape[0]
    x = x_ref[...]
    wg = wg_ref[...]
    logits = jnp.dot(x, wg, preferred_element_type=jnp.float32)

    lane = lax.broadcasted_iota(jnp.int32, (T, E), 1)
    m1 = jnp.max(logits, axis=1, keepdims=True)
    i1 = jnp.min(jnp.where(logits == m1, lane, E), axis=1, keepdims=True)
    masked = jnp.where(lane == i1, -jnp.inf, logits)
    m2 = jnp.max(masked, axis=1, keepdims=True)
    i2 = jnp.min(jnp.where(masked == m2, lane, E), axis=1, keepdims=True)

    g0 = 1.0 / (1.0 + jnp.exp(m2 - m1))
    g1 = 1.0 / (1.0 + jnp.exp(m1 - m2))

    A = (lane == i1).astype(jnp.float32)
    Bh = (lane == i2).astype(jnp.float32)

    C = jnp.concatenate([A, Bh], axis=1)
    acc = C
    s = 1
    while s < T:
        shifted = jnp.concatenate(
            [jnp.zeros((s, 2 * E), jnp.float32), acc[: T - s]], axis=0)
        acc = acc + shifted
        s *= 2
    ec = acc - C
    ecA = ec[:, :E]
    ecB = ec[:, E:]

    pos0 = jnp.sum(A * (ecA + ecB), axis=1, keepdims=True)
    pos1 = jnp.sum(Bh * (ecA + ecB + A), axis=1, keepdims=True)
    pos0i = pos0.astype(jnp.int32)
    pos1i = pos1.astype(jnp.int32)
    keep0 = pos0i < CAP
    keep1 = pos1i < CAP

    capl = lax.broadcasted_iota(jnp.int32, (T, CAP), 1)
    P0 = ((capl == pos0i) & keep0).astype(jnp.float32)
    P1 = ((capl == pos1i) & keep1).astype(jnp.float32)

    tcol = lax.broadcasted_iota(jnp.int32, (T, 1), 0).astype(jnp.float32)
    hi = lax.Precision.HIGHEST
    dn = (((0,), (0,)), ((), ()))
    tok = (lax.dot_general(A, P0 * tcol, dn, precision=hi)
           + lax.dot_general(Bh, P1 * tcol, dn, precision=hi))
    gslot = (lax.dot_general(A, P0 * g0, dn, precision=hi)
             + lax.dot_general(Bh, P1 * g1, dn, precision=hi))

    count = (lax.dot_general(A + Bh, jnp.ones((T, 1), jnp.float32), dn,
                             precision=hi)).astype(jnp.int32)
    cape = lax.broadcasted_iota(jnp.int32, (E, CAP), 1)
    eid = lax.broadcasted_iota(jnp.int32, (E, CAP), 0)
    spread = (eid * CAP + cape) & (T - 1)
    toki = jnp.where(cape < count, tok.astype(jnp.int32), spread)

    tok_slot_ref[...] = toki
    gate_slot_ref[...] = jnp.concatenate(
        [gslot, jnp.zeros((1, CAP), jnp.float32)], axis=0)

    sent = E * CAP
    s0_ref[...] = jnp.where(keep0, i1 * CAP + pos0i, sent)
    s1_ref[...] = jnp.where(keep1, i2 * CAP + pos1i, sent)


def _router_call(x, Wg):
    T = x.shape[0]
    return pl.pallas_call(
        _router_body,
        out_shape=(
            jax.ShapeDtypeStruct((E, CAP), jnp.int32),
            jax.ShapeDtypeStruct((E + 1, CAP), jnp.float32),
            jax.ShapeDtypeStruct((T, 1), jnp.int32),
            jax.ShapeDtypeStruct((T, 1), jnp.int32),
        ),
    )(x, Wg)


def _make_dispatch(T, D):
    SLOTS = E * CAP
    per_w = SLOTS // NW
    CH = 64
    NB = per_w // CH
    mesh = plsc.VectorSubcoreMesh(core_axis_name="c", subcore_axis_name="s",
                                  num_cores=NC, num_subcores=NS)

    @functools.partial(
        pl.kernel,
        out_type=jax.ShapeDtypeStruct((SLOTS, D), jnp.float32),
        mesh=mesh,
        scratch_types=[
            pltpu.VMEM((per_w,), jnp.int32),
            pltpu.VMEM((CH, D), jnp.float32),
            pltpu.VMEM((CH, D), jnp.float32),
            pltpu.SemaphoreType.DMA,
            pltpu.SemaphoreType.DMA,
        ],
    )
    def dispatch(x_hbm, tokslot_hbm, disp_hbm, idx_v, rows0_v, rows1_v,
                 sem0, sem1):
        wid = lax.axis_index("c") * NS + lax.axis_index("s")
        base = wid * per_w
        pltpu.sync_copy(tokslot_hbm.at[pl.ds(base, per_w)], idx_v)
        bufs = (rows0_v, rows1_v)
        sems = (sem0, sem1)
        cps = [None, None]
        cps[0] = pltpu.async_copy(
            x_hbm.at[idx_v.at[pl.ds(0, CH)]], rows0_v, sem0)
        for j in range(NB):
            p = j % 2
            cps[p].wait()
            if j + 1 < NB:
                q = (j + 1) % 2
                cps[q] = pltpu.async_copy(
                    x_hbm.at[idx_v.at[pl.ds((j + 1) * CH, CH)]],
                    bufs[q], sems[q])
            pltpu.sync_copy(bufs[p], disp_hbm.at[pl.ds(base + j * CH, CH)])

    return dispatch


def _ffn_body(disp_ref, w1_ref, b1_ref, w2_ref, b2_ref, gate_ref, y_ref):
    e = pl.program_id(0)
    f = pl.program_id(1)
    ec = jnp.minimum(e, E - 1)
    xb = disp_ref[...]
    h = jnp.dot(xb, w1_ref[0], preferred_element_type=jnp.float32)
    h = jax.nn.gelu(h + b1_ref[pl.ds(ec, 1), 0])
    yp = jnp.dot(h, w2_ref[0], preferred_element_type=jnp.float32)
    gcol = gate_ref[pl.ds(e, 1)].reshape(CAP, 1)

    @pl.when(f == 0)
    def _():
        y_ref[...] = (yp + b2_ref[pl.ds(ec, 1)]) * gcol

    @pl.when(f != 0)
    def _():
        y_ref[...] += yp * gcol


def _ffn_call(disp, W1, b1, W2, b2, gate_slot):
    D = disp.shape[1]
    F = W1.shape[2]
    FS = 2
    F2 = F // FS
    grid = (E + 1, FS)
    return pl.pallas_call(
        _ffn_body,
        grid=grid,
        in_specs=[
            pl.BlockSpec((CAP, D), lambda e, f: (jnp.minimum(e, E - 1), 0)),
            pl.BlockSpec((1, D, F2),
                         lambda e, f: (jnp.minimum(e, E - 1), 0, f)),
            pl.BlockSpec((E, FS, F2), lambda e, f: (0, 0, 0)),
            pl.BlockSpec((1, F2, D),
                         lambda e, f: (jnp.minimum(e, E - 1), f, 0)),
            pl.BlockSpec((E, D), lambda e, f: (0, 0)),
            pl.BlockSpec((E + 1, CAP, 1), lambda e, f: (0, 0, 0)),
        ],
        out_specs=pl.BlockSpec((CAP, D), lambda e, f: (e, 0)),
        out_shape=jax.ShapeDtypeStruct(((E + 1) * CAP, D), jnp.float32),
    )(disp, W1, b1.reshape(E, FS, F2), W2, b2, gate_slot)


def _make_combine(T, D):
    per_w = T // NW
    LPR = D // 16
    mesh = plsc.VectorSubcoreMesh(core_axis_name="c", subcore_axis_name="s",
                                  num_cores=NC, num_subcores=NS)

    @functools.partial(
        pl.kernel,
        out_type=jax.ShapeDtypeStruct((T, D), jnp.float32),
        mesh=mesh,
        scratch_types=[
            pltpu.VMEM((per_w,), jnp.int32),
            pltpu.VMEM((per_w, D), jnp.float32),
            pltpu.VMEM((per_w, D), jnp.float32),
            pltpu.SemaphoreType.DMA,
        ],
    )
    def combine(y_hbm, s0_hbm, s1_hbm, out_hbm, idx_v, rowsA_v, rowsB_v, sem):
        wid = lax.axis_index("c") * NS + lax.axis_index("s")
        base = wid * per_w
        pltpu.sync_copy(s0_hbm.at[pl.ds(base, per_w)], idx_v)
        pltpu.async_copy(y_hbm.at[idx_v], rowsA_v, sem).wait()
        pltpu.sync_copy(s1_hbm.at[pl.ds(base, per_w)], idx_v)
        pltpu.async_copy(y_hbm.at[idx_v], rowsB_v, sem).wait()

        def row_add(i, _):
            for c in range(LPR):
                sl = pl.ds(c * 16, 16)
                rowsA_v[i, sl] = rowsA_v[i, sl] + rowsB_v[i, sl]
            return 0

        lax.fori_loop(0, per_w, row_add, 0)
        pltpu.sync_copy(rowsA_v, out_hbm.at[pl.ds(base, per_w)])

    return combine


def kernel(hidden, Wg, W1, b1, W2, b2):
    B, S, D = hidden.shape
    T = B * S
    x = hidden.reshape(T, D)

    tok_slot, gate_slot, s0, s1 = _router_call(x, Wg)
    disp = _make_dispatch(T, D)(x, tok_slot.reshape(-1))
    y = _ffn_call(disp, W1, b1, W2, b2, gate_slot.reshape(E + 1, CAP, 1))
    out = _make_combine(T, D)(y, s0.reshape(-1), s1.reshape(-1))
    return out.reshape(B, S, D)


# speedup vs baseline: 1.1272x; 1.1272x over previous
import functools

import jax
import jax.numpy as jnp
from jax import lax
from jax.experimental import pallas as pl
from jax.experimental.pallas import tpu as pltpu
from jax.experimental.pallas import tpu_sc as plsc

E = 64
TOPK = 2
CAP = 128
NC = 2
NS = 16
NW = NC * NS


def _router_body(x_ref, wg_ref, tok_slot_ref, gate_slot_ref, s0_ref, s1_ref):
    T = x_ref.shape[0]
    x = x_ref[...]
    wg = wg_ref[...]
    logits = jnp.dot(x, wg, preferred_element_type=jnp.float32)

    lane = lax.broadcasted_iota(jnp.int32, (T, E), 1)
    m1 = jnp.max(logits, axis=1, keepdims=True)
    i1 = jnp.min(jnp.where(logits == m1, lane, E), axis=1, keepdims=True)
    masked = jnp.where(lane == i1, -jnp.inf, logits)
    m2 = jnp.max(masked, axis=1, keepdims=True)
    i2 = jnp.min(jnp.where(masked == m2, lane, E), axis=1, keepdims=True)

    g0 = 1.0 / (1.0 + jnp.exp(m2 - m1))
    g1 = 1.0 / (1.0 + jnp.exp(m1 - m2))

    A = (lane == i1).astype(jnp.float32)
    Bh = (lane == i2).astype(jnp.float32)

    C = jnp.concatenate([A, Bh], axis=1)
    acc = C
    s = 1
    while s < T:
        shifted = jnp.concatenate(
            [jnp.zeros((s, 2 * E), jnp.float32), acc[: T - s]], axis=0)
        acc = acc + shifted
        s *= 2
    ec = acc - C
    ecA = ec[:, :E]
    ecB = ec[:, E:]

    pos0 = jnp.sum(A * (ecA + ecB), axis=1, keepdims=True)
    pos1 = jnp.sum(Bh * (ecA + ecB + A), axis=1, keepdims=True)
    pos0i = pos0.astype(jnp.int32)
    pos1i = pos1.astype(jnp.int32)
    keep0 = pos0i < CAP
    keep1 = pos1i < CAP

    capl = lax.broadcasted_iota(jnp.int32, (T, CAP), 1)
    P0 = ((capl == pos0i) & keep0).astype(jnp.float32)
    P1 = ((capl == pos1i) & keep1).astype(jnp.float32)

    tcol = lax.broadcasted_iota(jnp.int32, (T, 1), 0).astype(jnp.float32)
    hi = lax.Precision.HIGHEST
    dn = (((0,), (0,)), ((), ()))
    tok = (lax.dot_general(A, P0 * tcol, dn, precision=hi)
           + lax.dot_general(Bh, P1 * tcol, dn, precision=hi))
    gslot = (lax.dot_general(A, P0 * g0, dn, precision=hi)
             + lax.dot_general(Bh, P1 * g1, dn, precision=hi))

    count = (lax.dot_general(A + Bh, jnp.ones((T, 1), jnp.float32), dn,
                             precision=hi)).astype(jnp.int32)
    cape = lax.broadcasted_iota(jnp.int32, (E, CAP), 1)
    eid = lax.broadcasted_iota(jnp.int32, (E, CAP), 0)
    spread = (eid * CAP + cape) & (T - 1)
    toki = jnp.where(cape < count, tok.astype(jnp.int32), spread)

    tok_slot_ref[...] = toki
    gate_slot_ref[...] = jnp.concatenate(
        [gslot, jnp.zeros((1, CAP), jnp.float32)], axis=0)

    sent = E * CAP
    s0_ref[...] = jnp.where(keep0, i1 * CAP + pos0i, sent)
    s1_ref[...] = jnp.where(keep1, i2 * CAP + pos1i, sent)


def _router_call(x, Wg):
    T = x.shape[0]
    return pl.pallas_call(
        _router_body,
        out_shape=(
            jax.ShapeDtypeStruct((E, CAP), jnp.int32),
            jax.ShapeDtypeStruct((E + 1, CAP), jnp.float32),
            jax.ShapeDtypeStruct((T, 1), jnp.int32),
            jax.ShapeDtypeStruct((T, 1), jnp.int32),
        ),
    )(x, Wg)


def _make_dispatch(T, D):
    SLOTS = E * CAP
    per_w = SLOTS // NW
    CH = 64
    NB = per_w // CH
    mesh = plsc.VectorSubcoreMesh(core_axis_name="c", subcore_axis_name="s",
                                  num_cores=NC, num_subcores=NS)

    @functools.partial(
        pl.kernel,
        out_type=jax.ShapeDtypeStruct((SLOTS, D), jnp.float32),
        mesh=mesh,
        scratch_types=[
            pltpu.VMEM((per_w,), jnp.int32),
            pltpu.VMEM((CH, D), jnp.float32),
            pltpu.VMEM((CH, D), jnp.float32),
            pltpu.SemaphoreType.DMA,
            pltpu.SemaphoreType.DMA,
        ],
    )
    def dispatch(x_hbm, tokslot_hbm, disp_hbm, idx_v, rows0_v, rows1_v,
                 sem0, sem1):
        wid = lax.axis_index("c") * NS + lax.axis_index("s")
        base = wid * per_w
        pltpu.sync_copy(tokslot_hbm.at[pl.ds(base, per_w)], idx_v)
        bufs = (rows0_v, rows1_v)
        sems = (sem0, sem1)
        cps = [None, None]
        cps[0] = pltpu.async_copy(
            x_hbm.at[idx_v.at[pl.ds(0, CH)]], rows0_v, sem0)
        for j in range(NB):
            p = j % 2
            cps[p].wait()
            if j + 1 < NB:
                q = (j + 1) % 2
                cps[q] = pltpu.async_copy(
                    x_hbm.at[idx_v.at[pl.ds((j + 1) * CH, CH)]],
                    bufs[q], sems[q])
            pltpu.sync_copy(bufs[p], disp_hbm.at[pl.ds(base + j * CH, CH)])

    return dispatch


def _ffn_body(disp_ref, w1_ref, b1_ref, w2_ref, b2_ref, gate_ref, y_ref):
    e = pl.program_id(0)
    ec = jnp.minimum(e, E - 1)
    xb = disp_ref[...]
    h = jnp.dot(xb, w1_ref[0], preferred_element_type=jnp.float32)
    h = jax.nn.gelu(h + b1_ref[pl.ds(ec, 1)])
    y = jnp.dot(h, w2_ref[0], preferred_element_type=jnp.float32)
    gcol = gate_ref[pl.ds(e, 1)].reshape(CAP, 1)
    y = (y + b2_ref[pl.ds(ec, 1)]) * gcol
    y_ref[...] = y


def _ffn_call(disp, W1, b1, W2, b2, gate_slot):
    D = disp.shape[1]
    F = W1.shape[2]
    grid = (E + 1,)
    ew3 = lambda e: (jnp.minimum(e, E - 1), 0, 0)
    zm = lambda e: (0, 0)
    return pl.pallas_call(
        _ffn_body,
        grid=grid,
        in_specs=[
            pl.BlockSpec((CAP, D), lambda e: (jnp.minimum(e, E - 1), 0)),
            pl.BlockSpec((1, D, F), ew3),
            pl.BlockSpec((E, F), zm),
            pl.BlockSpec((1, F, D), ew3),
            pl.BlockSpec((E, D), zm),
            pl.BlockSpec((E + 1, CAP, 1), lambda e: (0, 0, 0)),
        ],
        out_specs=pl.BlockSpec((CAP, D), lambda e: (e, 0)),
        out_shape=jax.ShapeDtypeStruct(((E + 1) * CAP, D), jnp.float32),
    )(disp, W1, b1, W2, b2, gate_slot)


def _make_combine(T, D):
    per_w = T // NW
    LPR = D // 16
    mesh = plsc.VectorSubcoreMesh(core_axis_name="c", subcore_axis_name="s",
                                  num_cores=NC, num_subcores=NS)

    @functools.partial(
        pl.kernel,
        out_type=jax.ShapeDtypeStruct((T, D), jnp.float32),
        mesh=mesh,
        scratch_types=[
            pltpu.VMEM((per_w,), jnp.int32),
            pltpu.VMEM((per_w, D), jnp.float32),
            pltpu.VMEM((per_w, D), jnp.float32),
            pltpu.SemaphoreType.DMA,
        ],
    )
    def combine(y_hbm, s0_hbm, s1_hbm, out_hbm, idx_v, rowsA_v, rowsB_v, sem):
        wid = lax.axis_index("c") * NS + lax.axis_index("s")
        base = wid * per_w
        pltpu.sync_copy(s0_hbm.at[pl.ds(base, per_w)], idx_v)
        pltpu.async_copy(y_hbm.at[idx_v], rowsA_v, sem).wait()
        pltpu.sync_copy(s1_hbm.at[pl.ds(base, per_w)], idx_v)
        pltpu.async_copy(y_hbm.at[idx_v], rowsB_v, sem).wait()

        def row_add(i, _):
            for c in range(LPR):
                sl = pl.ds(c * 16, 16)
                rowsA_v[i, sl] = rowsA_v[i, sl] + rowsB_v[i, sl]
            return 0

        lax.fori_loop(0, per_w, row_add, 0)
        pltpu.sync_copy(rowsA_v, out_hbm.at[pl.ds(base, per_w)])

    return combine


def kernel(hidden, Wg, W1, b1, W2, b2):
    B, S, D = hidden.shape
    T = B * S
    x = hidden.reshape(T, D)

    tok_slot, gate_slot, s0, s1 = _router_call(x, Wg)
    disp = _make_dispatch(T, D)(x, tok_slot.reshape(-1))
    y = _ffn_call(disp, W1, b1, W2, b2, gate_slot.reshape(E + 1, CAP, 1))
    out = _make_combine(T, D)(y, s0.reshape(-1), s1.reshape(-1))
    return out.reshape(B, S, D)


# speedup vs baseline: 1.1294x; 1.0019x over previous
import functools

import jax
import jax.numpy as jnp
from jax import lax
from jax.experimental import pallas as pl
from jax.experimental.pallas import tpu as pltpu
from jax.experimental.pallas import tpu_sc as plsc

E = 64
TOPK = 2
CAP = 128
NC = 2
NS = 16
NW = NC * NS


def _router_body(x_ref, wg_ref, tok_slot_ref, gate_slot_ref, s0_ref, s1_ref):
    T = x_ref.shape[0]
    x = x_ref[...]
    wg = wg_ref[...]
    logits = jnp.dot(x, wg, preferred_element_type=jnp.float32)

    lane = lax.broadcasted_iota(jnp.int32, (T, E), 1)
    m1 = jnp.max(logits, axis=1, keepdims=True)
    i1 = jnp.min(jnp.where(logits == m1, lane, E), axis=1, keepdims=True)
    masked = jnp.where(lane == i1, -jnp.inf, logits)
    m2 = jnp.max(masked, axis=1, keepdims=True)
    i2 = jnp.min(jnp.where(masked == m2, lane, E), axis=1, keepdims=True)

    g0 = 1.0 / (1.0 + jnp.exp(m2 - m1))
    g1 = 1.0 / (1.0 + jnp.exp(m1 - m2))

    A = (lane == i1).astype(jnp.float32)
    Bh = (lane == i2).astype(jnp.float32)

    C = jnp.concatenate([A, Bh], axis=1)
    acc = C
    s = 1
    while s < T:
        shifted = jnp.concatenate(
            [jnp.zeros((s, 2 * E), jnp.float32), acc[: T - s]], axis=0)
        acc = acc + shifted
        s *= 2
    ec = acc - C
    ecA = ec[:, :E]
    ecB = ec[:, E:]

    pos0 = jnp.sum(A * (ecA + ecB), axis=1, keepdims=True)
    pos1 = jnp.sum(Bh * (ecA + ecB + A), axis=1, keepdims=True)
    pos0i = pos0.astype(jnp.int32)
    pos1i = pos1.astype(jnp.int32)
    keep0 = pos0i < CAP
    keep1 = pos1i < CAP

    capl = lax.broadcasted_iota(jnp.int32, (T, CAP), 1)
    P0 = ((capl == pos0i) & keep0).astype(jnp.float32)
    P1 = ((capl == pos1i) & keep1).astype(jnp.float32)

    tcol = lax.broadcasted_iota(jnp.int32, (T, 1), 0).astype(jnp.float32)
    hi = lax.Precision.HIGHEST
    dn = (((0,), (0,)), ((), ()))
    tok = (lax.dot_general(A, P0 * tcol, dn, precision=hi)
           + lax.dot_general(Bh, P1 * tcol, dn, precision=hi))
    gslot = (lax.dot_general(A, P0 * g0, dn, precision=hi)
             + lax.dot_general(Bh, P1 * g1, dn, precision=hi))

    count = (lax.dot_general(A + Bh, jnp.ones((T, 1), jnp.float32), dn,
                             precision=hi)).astype(jnp.int32)
    cape = lax.broadcasted_iota(jnp.int32, (E, CAP), 1)
    eid = lax.broadcasted_iota(jnp.int32, (E, CAP), 0)
    spread = (eid * CAP + cape) & (T - 1)
    toki = jnp.where(cape < count, tok.astype(jnp.int32), spread)

    tok_slot_ref[...] = toki
    gate_slot_ref[...] = jnp.concatenate(
        [gslot, jnp.zeros((1, CAP), jnp.float32)], axis=0)

    sent = E * CAP
    s0_ref[...] = jnp.where(keep0, i1 * CAP + pos0i, sent)
    s1_ref[...] = jnp.where(keep1, i2 * CAP + pos1i, sent)


def _router_call(x, Wg):
    T = x.shape[0]
    return pl.pallas_call(
        _router_body,
        out_shape=(
            jax.ShapeDtypeStruct((E, CAP), jnp.int32),
            jax.ShapeDtypeStruct((E + 1, CAP), jnp.float32),
            jax.ShapeDtypeStruct((T, 1), jnp.int32),
            jax.ShapeDtypeStruct((T, 1), jnp.int32),
        ),
    )(x, Wg)


def _make_dispatch(T, D):
    SLOTS = E * CAP
    per_w = SLOTS // NW
    CH = 64
    NB = per_w // CH
    mesh = plsc.VectorSubcoreMesh(core_axis_name="c", subcore_axis_name="s",
                                  num_cores=NC, num_subcores=NS)

    @functools.partial(
        pl.kernel,
        out_type=jax.ShapeDtypeStruct((SLOTS, D), jnp.float32),
        mesh=mesh,
        scratch_types=[
            pltpu.VMEM((per_w,), jnp.int32),
            pltpu.VMEM((CH, D), jnp.float32),
            pltpu.VMEM((CH, D), jnp.float32),
            pltpu.SemaphoreType.DMA,
            pltpu.SemaphoreType.DMA,
        ],
    )
    def dispatch(x_hbm, tokslot_hbm, disp_hbm, idx_v, rows0_v, rows1_v,
                 sem0, sem1):
        wid = lax.axis_index("c") * NS + lax.axis_index("s")
        base = wid * per_w
        pltpu.sync_copy(tokslot_hbm.at[pl.ds(base, per_w)], idx_v)
        bufs = (rows0_v, rows1_v)
        sems = (sem0, sem1)
        cps = [None, None]
        cps[0] = pltpu.async_copy(
            x_hbm.at[idx_v.at[pl.ds(0, CH)]], rows0_v, sem0)
        for j in range(NB):
            p = j % 2
            cps[p].wait()
            if j + 1 < NB:
                q = (j + 1) % 2
                cps[q] = pltpu.async_copy(
                    x_hbm.at[idx_v.at[pl.ds((j + 1) * CH, CH)]],
                    bufs[q], sems[q])
            pltpu.sync_copy(bufs[p], disp_hbm.at[pl.ds(base + j * CH, CH)])

    return dispatch


def _ffn_body(disp_ref, w1_ref, b1_ref, w2_ref, b2_ref, gate_ref, y_ref):
    e = pl.program_id(0)
    ec = jnp.minimum(e, E - 1)
    xb = disp_ref[...]
    h = jnp.dot(xb, w1_ref[0], preferred_element_type=jnp.float32)
    h = jax.nn.gelu(h + b1_ref[pl.ds(ec, 1)])
    y = jnp.dot(h, w2_ref[0], preferred_element_type=jnp.float32)
    gcol = gate_ref[pl.ds(e, 1)].reshape(CAP, 1)
    y = (y + b2_ref[pl.ds(ec, 1)]) * gcol
    y_ref[...] = y


def _ffn_call(disp, W1, b1, W2, b2, gate_slot):
    D = disp.shape[1]
    F = W1.shape[2]
    grid = (E + 1,)
    ew3 = lambda e: (jnp.minimum(e, E - 1), 0, 0)
    zm = lambda e: (0, 0)
    return pl.pallas_call(
        _ffn_body,
        grid=grid,
        in_specs=[
            pl.BlockSpec((CAP, D), lambda e: (jnp.minimum(e, E - 1), 0)),
            pl.BlockSpec((1, D, F), ew3),
            pl.BlockSpec((E, F), zm),
            pl.BlockSpec((1, F, D), ew3),
            pl.BlockSpec((E, D), zm),
            pl.BlockSpec((E + 1, CAP, 1), lambda e: (0, 0, 0)),
        ],
        out_specs=pl.BlockSpec((CAP, D), lambda e: (e, 0)),
        out_shape=jax.ShapeDtypeStruct(((E + 1) * CAP, D), jnp.float32),
    )(disp, W1, b1, W2, b2, gate_slot)


def _make_combine(T, D):
    per_w = T // NW
    LPR = D // 16
    mesh = plsc.VectorSubcoreMesh(core_axis_name="c", subcore_axis_name="s",
                                  num_cores=NC, num_subcores=NS)

    @functools.partial(
        pl.kernel,
        out_type=jax.ShapeDtypeStruct((T, D), jnp.float32),
        mesh=mesh,
        scratch_types=[
            pltpu.VMEM((per_w,), jnp.int32),
            pltpu.VMEM((per_w,), jnp.int32),
            pltpu.VMEM((per_w, D), jnp.float32),
            pltpu.VMEM((per_w, D), jnp.float32),
            pltpu.SemaphoreType.DMA,
            pltpu.SemaphoreType.DMA,
        ],
    )
    def combine(y_hbm, s0_hbm, s1_hbm, out_hbm, idxA_v, idxB_v,
                rowsA_v, rowsB_v, semA, semB):
        wid = lax.axis_index("c") * NS + lax.axis_index("s")
        base = wid * per_w
        pltpu.sync_copy(s0_hbm.at[pl.ds(base, per_w)], idxA_v)
        cpA = pltpu.async_copy(y_hbm.at[idxA_v], rowsA_v, semA)
        pltpu.sync_copy(s1_hbm.at[pl.ds(base, per_w)], idxB_v)
        cpB = pltpu.async_copy(y_hbm.at[idxB_v], rowsB_v, semB)
        cpA.wait()
        cpB.wait()

        def row_add(i, _):
            for c in range(LPR):
                sl = pl.ds(c * 16, 16)
                rowsA_v[i, sl] = rowsA_v[i, sl] + rowsB_v[i, sl]
            return 0

        lax.fori_loop(0, per_w, row_add, 0)
        pltpu.sync_copy(rowsA_v, out_hbm.at[pl.ds(base, per_w)])

    return combine


def kernel(hidden, Wg, W1, b1, W2, b2):
    B, S, D = hidden.shape
    T = B * S
    x = hidden.reshape(T, D)

    tok_slot, gate_slot, s0, s1 = _router_call(x, Wg)
    disp = _make_dispatch(T, D)(x, tok_slot.reshape(-1))
    y = _ffn_call(disp, W1, b1, W2, b2, gate_slot.reshape(E + 1, CAP, 1))
    out = _make_combine(T, D)(y, s0.reshape(-1), s1.reshape(-1))
    return out.reshape(B, S, D)
